# baseline (speedup 1.0000x reference)
import functools

import jax
import jax.numpy as jnp
from jax import lax
from jax.experimental import pallas as pl
from jax.experimental.pallas import tpu as pltpu
from jax.experimental.pallas import tpu_sc as plsc

NC = 2
NS = 16
NW = NC * NS
LANES = 16

BATCH = 16384
FEAT = 64
B_PER_W = BATCH // NW
CHUNK = 128
NCHUNK = B_PER_W // CHUNK
QUEUES = 4


def _make_sc_kernel():
    mesh = plsc.VectorSubcoreMesh(core_axis_name="c", subcore_axis_name="s")

    @functools.partial(
        pl.kernel,
        mesh=mesh,
        out_type=jax.ShapeDtypeStruct((NW, LANES), jnp.float32),
        scratch_types=[
            pltpu.VMEM((NCHUNK, CHUNK), jnp.int32),
            pltpu.VMEM((2, CHUNK, FEAT), jnp.float32),
            pltpu.VMEM((B_PER_W, FEAT), jnp.float32),
            pltpu.VMEM((LANES,), jnp.float32),
        ] + [pltpu.SemaphoreType.DMA] * (2 * QUEUES),
    )
    def body(x_hbm, lab_hbm, table_hbm, out_hbm,
             lab_v, rows_v, x_v, acc_v, *sems):
        wid = lax.axis_index("s") * NC + lax.axis_index("c")
        base = wid * B_PER_W

        pltpu.sync_copy(lab_hbm.at[wid], lab_v)

        def issue_chunk(k):
            buf = k % 2

            def g_body(g, carry):
                lvec = lab_v[k, pl.ds(g * LANES, LANES)]
                for j in range(LANES):
                    s = lvec[j]
                    pltpu.async_copy(
                        table_hbm.at[s],
                        rows_v.at[buf, g * LANES + j],
                        sems[buf * QUEUES + (j % QUEUES)],
                    )
                return carry

            lax.fori_loop(0, CHUNK // LANES, g_body, 0)

        def drain_chunk(k):
            buf = k % 2
            per_q = CHUNK // QUEUES
            for q in range(QUEUES):
                pltpu.make_async_copy(
                    table_hbm.at[pl.ds(0, per_q)],
                    rows_v.at[buf, pl.ds(0, per_q)],
                    sems[buf * QUEUES + q],
                ).wait()

        issue_chunk(0)
        issue_chunk(1)
        pltpu.sync_copy(x_hbm.at[pl.ds(base, B_PER_W)], x_v)

        def chunk_sum(k, acc):
            buf = k % 2

            def row(r, acc):
                for c in range(FEAT // LANES):
                    xa = x_v[k * CHUNK + r, pl.ds(c * LANES, LANES)]
                    ga = rows_v[buf, r, pl.ds(c * LANES, LANES)]
                    d = xa - ga
                    acc = acc + d * d
                return acc

            return lax.fori_loop(0, CHUNK, row, acc)

        acc = jnp.zeros((LANES,), jnp.float32)
        for k in range(NCHUNK):
            drain_chunk(k)
            acc = chunk_sum(k, acc)
            if k + 2 < NCHUNK:
                issue_chunk(k + 2)
        acc_v[...] = acc
        pltpu.sync_copy(acc_v, out_hbm.at[wid])

    return body


_sc_loss_partials = _make_sc_kernel()


@jax.jit
def kernel(x, labels, centers):
    batch, feat = x.shape
    lab = labels.astype(jnp.int32).reshape(NW, NCHUNK, CHUNK)
    partials = _sc_loss_partials(x, lab, centers)
    return jnp.sum(partials) / 2.0 / batch
